# baseline (speedup 1.0000x reference)
import functools

import jax
import jax.numpy as jnp
from jax import lax
from jax.experimental import pallas as pl
from jax.experimental.pallas import tpu as pltpu
from jax.experimental.pallas import tpu_sc as plsc

D = 768
B = 4
T = 4096

_info = plsc.get_sparse_core_info()
NC, NS, L = _info.num_cores, _info.num_subcores, _info.num_lanes
NW = NC * NS
PW_T = T // NW
CH = 32
NCHUNK = PW_T // CH
NSTEP = NCHUNK * B
NRING = 3


def _emb_body(tok_hbm, x_hbm, pos_hbm, out_hbm, idx_v, rows, pos_v, gsem, ssem, psem, isem):
    wid = lax.axis_index("s") * NC + lax.axis_index("c")
    t0 = wid * PW_T

    icopy = [
        pltpu.async_copy(x_hbm.at[b, pl.ds(t0, PW_T)], idx_v.at[b], isem)
        for b in range(B)
    ]

    def start_gather(s, k):
        c = s // B
        b = s - c * B
        return pltpu.async_copy(
            tok_hbm.at[idx_v.at[b, pl.ds(c * CH, CH)]], rows[k], gsem[k])

    def start_pos(c, half):
        return pltpu.async_copy(
            pos_hbm.at[pl.ds(t0 + c * CH, CH)], pos_v.at[pl.ds(half * CH, CH)],
            psem)

    def start_store(s, k):
        c = s // B
        b = s - c * B
        return pltpu.async_copy(
            rows[k], out_hbm.at[b, pl.ds(t0 + c * CH, CH)], ssem[k])

    pdesc = start_pos(0, 0)
    for cp in icopy:
        cp.wait()
    gdesc = [start_gather(0, 0), start_gather(1, 1), None]
    sdesc = [None] * NRING

    def run_step(s, k):
        c = s // B
        b = s - c * B
        q = lax.rem(c, 2)
        gdesc[k].wait()

        @pl.when(b == 0)
        def _():
            pdesc.wait()

            @pl.when(c + 1 < NCHUNK)
            def _():
                start_pos(c + 1, 1 - q)

        def row_body(r, carry, _k=k):
            pr = q * CH + r
            for j in range(D // L):
                sl = pl.ds(j * L, L)
                rows[_k][r, sl] = rows[_k][r, sl] + pos_v[pr, sl]
            return carry

        lax.fori_loop(0, CH, row_body, 0)
        start_store(s, k)

        g = s + 2
        kg = (k + 2) % NRING

        @pl.when(g < NSTEP)
        def _():
            @pl.when(s >= 1)
            def _():
                sdesc[kg].wait()

            start_gather(g, kg)

    sdesc = [
        pltpu.make_async_copy(rows[k], out_hbm.at[0, pl.ds(0, CH)], ssem[k])
        for k in range(NRING)
    ]
    gdesc = [
        pltpu.make_async_copy(
            tok_hbm.at[idx_v.at[0, pl.ds(0, CH)]], rows[k], gsem[k])
        for k in range(NRING)
    ]

    @pl.loop(0, NSTEP - 1, step=NRING)
    def _(s0):
        for i in range(NRING):
            run_step(s0 + i, i)

    run_step(jnp.int32(NSTEP - 1), (NSTEP - 1) % NRING)

    for k in range(NRING):
        sdesc[k].wait()


@functools.partial(
    pl.kernel,
    mesh=plsc.VectorSubcoreMesh(core_axis_name="c", subcore_axis_name="s"),
    out_type=jax.ShapeDtypeStruct((B, T, D), jnp.float32),
    scratch_types=[
        pltpu.VMEM((B, PW_T), jnp.int32),
        [pltpu.VMEM((CH, D), jnp.float32) for _ in range(NRING)],
        pltpu.VMEM((2 * CH, D), jnp.float32),
        [pltpu.SemaphoreType.DMA for _ in range(NRING)],
        [pltpu.SemaphoreType.DMA for _ in range(NRING)],
        pltpu.SemaphoreType.DMA,
        pltpu.SemaphoreType.DMA,
    ],
)
def _emb_kernel(tok_hbm, x_hbm, pos_hbm, out_hbm, idx_v, rows, pos_v, gsem, ssem, psem, isem):
    _emb_body(tok_hbm, x_hbm, pos_hbm, out_hbm, idx_v, rows, pos_v, gsem, ssem, psem, isem)


def kernel(x, tok_table, pos_table):
    return _emb_kernel(tok_table, x.astype(jnp.int32), pos_table)


# speedup vs baseline: 1.9923x; 1.9923x over previous
import functools

import jax
import jax.numpy as jnp
from jax import lax
from jax.experimental import pallas as pl
from jax.experimental.pallas import tpu as pltpu
from jax.experimental.pallas import tpu_sc as plsc

D = 768
B = 4
T = 4096

_info = plsc.get_sparse_core_info()
NC, NS, L = _info.num_cores, _info.num_subcores, _info.num_lanes
NW = NC * NS
PW_T = T // NW
CH = 32
NCHUNK = PW_T // CH
NSTEP = NCHUNK * B
NRING = 3


def _emb_body(tok_hbm, x_hbm, pos_hbm, out_hbm, idx_v, rows, pos, gsem, ssem, psem, isem):
    wid = lax.axis_index("s") * NC + lax.axis_index("c")
    t0 = wid * PW_T

    icopy = [
        pltpu.async_copy(x_hbm.at[b, pl.ds(t0, PW_T)], idx_v.at[b], isem)
        for b in range(B)
    ]

    def start_gather(s, k):
        c, b = s // B, s % B
        return pltpu.async_copy(
            tok_hbm.at[idx_v.at[b, pl.ds(c * CH, CH)]], rows[k], gsem[k])

    pcopy = [None] * 2
    pcopy[0] = pltpu.async_copy(pos_hbm.at[pl.ds(t0, CH)], pos[0], psem[0])
    for c in icopy:
        c.wait()
    gcopy = [None] * NRING
    scopy = [None] * NRING
    gcopy[0] = start_gather(0, 0)
    gcopy[1] = start_gather(1, 1)

    for s in range(NSTEP):
        k = s % NRING
        c, b = s // B, s % B
        q = c % 2
        gcopy[k].wait()
        if b == 0:
            pcopy[q].wait()
            if c + 1 < NCHUNK:
                pcopy[1 - q] = pltpu.async_copy(
                    pos_hbm.at[pl.ds(t0 + (c + 1) * CH, CH)], pos[1 - q], psem[1 - q])

        def row_body(r, carry, _k=k, _q=q):
            for j in range(D // L):
                sl = pl.ds(j * L, L)
                rows[_k][r, sl] = rows[_k][r, sl] + pos[_q][r, sl]
            return carry

        lax.fori_loop(0, CH, row_body, 0)

        scopy[k] = pltpu.async_copy(
            rows[k], out_hbm.at[b, pl.ds(t0 + c * CH, CH)], ssem[k])

        g = s + 2
        if g < NSTEP:
            kg = g % NRING
            if scopy[kg] is not None:
                scopy[kg].wait()
                scopy[kg] = None
            gcopy[kg] = start_gather(g, kg)

    for k in range(NRING):
        if scopy[k] is not None:
            scopy[k].wait()


@functools.partial(
    pl.kernel,
    mesh=plsc.VectorSubcoreMesh(core_axis_name="c", subcore_axis_name="s"),
    out_type=jax.ShapeDtypeStruct((B, T, D), jnp.float32),
    scratch_types=[
        pltpu.VMEM((B, PW_T), jnp.int32),
        [pltpu.VMEM((CH, D), jnp.float32) for _ in range(NRING)],
        [pltpu.VMEM((CH, D), jnp.float32) for _ in range(2)],
        [pltpu.SemaphoreType.DMA for _ in range(NRING)],
        [pltpu.SemaphoreType.DMA for _ in range(NRING)],
        [pltpu.SemaphoreType.DMA for _ in range(2)],
        pltpu.SemaphoreType.DMA,
    ],
)
def _emb_kernel(tok_hbm, x_hbm, pos_hbm, out_hbm, idx_v, rows, pos, gsem, ssem, psem, isem):
    _emb_body(tok_hbm, x_hbm, pos_hbm, out_hbm, idx_v, rows, pos, gsem, ssem, psem, isem)


def kernel(x, tok_table, pos_table):
    return _emb_kernel(tok_table, x.astype(jnp.int32), pos_table)
